# Initial kernel scaffold; baseline (speedup 1.0000x reference)
import jax
import jax.numpy as jnp
from jax.experimental import pallas as pl


def kernel(x, edge_index, Wl1, Wr1, att1, b1, Wl2, Wr2, att2, b2):
    raise NotImplementedError("write your pallas kernel here")


# speedup vs baseline: 9.0428x; 9.0428x over previous
import functools

import jax
import jax.numpy as jnp
from jax import lax
from jax.experimental import pallas as pl
from jax.experimental.pallas import tpu as pltpu
from jax.experimental.pallas import tpu_sc as plsc

N_NODES = 10000
N_PAD = 10112
F_IN = 128
H1, C = 8, 32
D1 = H1 * C
E_RAW = 320000
E_TOT = E_RAW + N_NODES
K_CHUNK = 128
E_PAD = 344064
ACC_W = 48
ROWS_PER_TILE = N_PAD // 16

_NSC = 2
_NTILES = 16

_Z16 = None


_R1 = 2528
_R3 = 2000


def _proj1_body(x_ref, wl_ref, wr_ref, xl_ref, xr_ref):
    xb = x_ref[...]
    dn = (((1,), (1,)), ((), ()))
    xl_ref[0] = lax.dot_general(xb, wl_ref[0], dn,
                                preferred_element_type=jnp.float32)
    xr_ref[0] = lax.dot_general(xb, wr_ref[0], dn,
                                preferred_element_type=jnp.float32)


def _proj1(x_pad, wl3, wr3):
    grid = (H1, N_PAD // _R1)
    return pl.pallas_call(
        _proj1_body,
        grid=grid,
        in_specs=[
            pl.BlockSpec((_R1, F_IN), lambda h, r: (r, 0)),
            pl.BlockSpec((1, C, F_IN), lambda h, r: (h, 0, 0)),
            pl.BlockSpec((1, C, F_IN), lambda h, r: (h, 0, 0)),
        ],
        out_specs=[
            pl.BlockSpec((1, _R1, C), lambda h, r: (h, r, 0)),
            pl.BlockSpec((1, _R1, C), lambda h, r: (h, r, 0)),
        ],
        out_shape=[
            jax.ShapeDtypeStruct((H1, N_PAD, C), jnp.float32),
            jax.ShapeDtypeStruct((H1, N_PAD, C), jnp.float32),
        ],
    )(x_pad, wl3, wr3)


def _mid_body(acc_ref, b1_ref, wl2_ref, wr2_ref, xl2_ref, xr2_ref):
    parts = []
    for h in range(H1):
        num = acc_ref[h, :, 0:C]
        den = acc_ref[h, :, C:C + 1] + 1e-16
        parts.append(num / den)
    h1 = jnp.concatenate(parts, axis=1) + b1_ref[...]
    h1 = jax.nn.gelu(h1)
    dn = (((1,), (1,)), ((), ()))
    xl2_ref[...] = lax.dot_general(h1, wl2_ref[...], dn,
                                   preferred_element_type=jnp.float32)
    xr2_ref[...] = lax.dot_general(h1, wr2_ref[...], dn,
                                   preferred_element_type=jnp.float32)


def _mid(acc1, b1_2d, wl2, wr2):
    grid = (N_PAD // _R1,)
    return pl.pallas_call(
        _mid_body,
        grid=grid,
        in_specs=[
            pl.BlockSpec((H1, _R1, ACC_W), lambda r: (0, r, 0)),
            pl.BlockSpec((1, D1), lambda r: (0, 0)),
            pl.BlockSpec((C, D1), lambda r: (0, 0)),
            pl.BlockSpec((C, D1), lambda r: (0, 0)),
        ],
        out_specs=[
            pl.BlockSpec((_R1, C), lambda r: (r, 0)),
            pl.BlockSpec((_R1, C), lambda r: (r, 0)),
        ],
        out_shape=[
            jax.ShapeDtypeStruct((N_PAD, C), jnp.float32),
            jax.ShapeDtypeStruct((N_PAD, C), jnp.float32),
        ],
    )(acc1, b1_2d, wl2, wr2)


def _final_body(acc_ref, b2_ref, out_ref):
    a = acc_ref[0] + acc_ref[1]
    num = a[:, 0:C]
    den = a[:, C:C + 1] + 1e-16
    out_ref[...] = num / den + b2_ref[...]


def _final(acc2, b2_2d):
    grid = (N_NODES // _R3,)
    return pl.pallas_call(
        _final_body,
        grid=grid,
        in_specs=[
            pl.BlockSpec((_NSC, _R3, ACC_W), lambda r: (0, r, 0)),
            pl.BlockSpec((1, C), lambda r: (0, 0)),
        ],
        out_specs=pl.BlockSpec((_R3, C), lambda r: (r, 0)),
        out_shape=jax.ShapeDtypeStruct((N_NODES, C), jnp.float32),
    )(acc2, b2_2d)


_MESH = plsc.VectorSubcoreMesh(core_axis_name="c", subcore_axis_name="s")
_SC_PARAMS = pltpu.CompilerParams(use_tc_tiling_on_sc=False,
                                  needs_layout_passes=False)


def _zero_scratch(zbuf, contrib):
    z16 = jnp.zeros((16,), jnp.float32)

    def zb(i, _):
        for j in range(ACC_W // 16):
            zbuf[i, pl.ds(j * 16, 16)] = z16
        return 0

    lax.fori_loop(0, ROWS_PER_TILE, zb, 0)

    def zc(i, _):
        contrib[i, pl.ds(C, 16)] = z16
        return 0

    lax.fori_loop(0, K_CHUNK, zc, 0)


def _edge_chunk(h_off, e0, src_ref, dst_ref, xl_flat, xr_flat,
                src_v, dst_v, srch_v, dsth_v, xl_s, xr_s, contrib,
                att_vm, acc_sh, sem1, sem2):
    pltpu.sync_copy(src_ref.at[pl.ds(e0, K_CHUNK)], src_v)
    pltpu.sync_copy(dst_ref.at[pl.ds(e0, K_CHUNK)], dst_v)

    def mkoff(i, _):
        srch_v[pl.ds(i * 16, 16)] = src_v[pl.ds(i * 16, 16)] + h_off
        dsth_v[pl.ds(i * 16, 16)] = dst_v[pl.ds(i * 16, 16)] + h_off
        return 0

    lax.fori_loop(0, K_CHUNK // 16, mkoff, 0)

    c1 = pltpu.async_copy(xl_flat.at[srch_v], xl_s, sem1)
    c2 = pltpu.async_copy(xr_flat.at[dsth_v], xr_s, sem2)
    c1.wait()
    c2.wait()

    iota = lax.iota(jnp.int32, 16)

    def group(g, _):
        rows = g * 16 + iota
        acc = jnp.zeros((16,), jnp.float32)
        a_vals = []
        for cc in range(C):
            colv = jnp.full((16,), cc, jnp.int32)
            a = plsc.load_gather(xl_s, [rows, colv])
            b = plsc.load_gather(xr_s, [rows, colv])
            s = a + b
            lr = jnp.where(s >= 0.0, s, 0.2 * s)
            acc = acc + att_vm[cc] * lr
            a_vals.append(a)
        ev = jnp.exp(jnp.minimum(acc, 45.0))
        plsc.store_scatter(contrib, [rows, jnp.full((16,), C, jnp.int32)], ev)
        for cc in range(C):
            colv = jnp.full((16,), cc, jnp.int32)
            plsc.store_scatter(contrib, [rows, colv], ev * a_vals[cc])
        return 0

    lax.fori_loop(0, K_CHUNK // 16, group, 0)

    pltpu.sync_copy(contrib, acc_sh.at[dst_v], add=True)


def _l1_body(xl_flat, xr_flat, att_ref, src_ref, dst_ref, out_ref,
             src_v, dst_v, srch_v, dsth_v, xl_s, xr_s, contrib, zbuf,
             att_vm, acc_sh, sem1, sem2):
    cid = lax.axis_index("c")
    sid = lax.axis_index("s")
    _zero_scratch(zbuf, contrib)
    row0 = sid * ROWS_PER_TILE
    edges_per_tile = E_PAD // _NTILES
    n_chunks = edges_per_tile // K_CHUNK

    def head(hh, _):
        h = cid * (H1 // _NSC) + hh
        pltpu.sync_copy(att_ref.at[pl.ds(h * C, C)], att_vm)
        pltpu.sync_copy(zbuf, acc_sh.at[pl.ds(row0, ROWS_PER_TILE)])
        plsc.subcore_barrier()
        h_off = h * N_PAD

        def chunk(k, _):
            e0 = sid * edges_per_tile + k * K_CHUNK
            _edge_chunk(h_off, e0, src_ref, dst_ref, xl_flat, xr_flat,
                        src_v, dst_v, srch_v, dsth_v, xl_s, xr_s, contrib,
                        att_vm, acc_sh, sem1, sem2)
            return 0

        lax.fori_loop(0, n_chunks, chunk, 0)
        plsc.subcore_barrier()
        pltpu.sync_copy(acc_sh.at[pl.ds(row0, ROWS_PER_TILE)],
                        out_ref.at[h].at[pl.ds(row0, ROWS_PER_TILE)])
        return 0

    lax.fori_loop(0, H1 // _NSC, head, 0)


_l1_edges = functools.partial(
    pl.kernel,
    out_type=jax.ShapeDtypeStruct((H1, N_PAD, ACC_W), jnp.float32),
    mesh=_MESH,
    compiler_params=_SC_PARAMS,
    scratch_types=[
        pltpu.VMEM((K_CHUNK,), jnp.int32),
        pltpu.VMEM((K_CHUNK,), jnp.int32),
        pltpu.VMEM((K_CHUNK,), jnp.int32),
        pltpu.VMEM((K_CHUNK,), jnp.int32),
        pltpu.VMEM((K_CHUNK, C), jnp.float32),
        pltpu.VMEM((K_CHUNK, C), jnp.float32),
        pltpu.VMEM((K_CHUNK, ACC_W), jnp.float32),
        pltpu.VMEM((ROWS_PER_TILE, ACC_W), jnp.float32),
        pltpu.VMEM((C, 16), jnp.float32),
        pltpu.VMEM_SHARED((N_PAD, ACC_W), jnp.float32),
        pltpu.SemaphoreType.DMA,
        pltpu.SemaphoreType.DMA,
    ],
)(_l1_body)


def _l2_body(xl2_ref, xr2_ref, att_ref, src_ref, dst_ref, out_ref,
             src_v, dst_v, xl_s, xr_s, contrib, zbuf,
             att_vm, acc_sh, sem1, sem2):
    cid = lax.axis_index("c")
    sid = lax.axis_index("s")
    _zero_scratch(zbuf, contrib)
    row0 = sid * ROWS_PER_TILE
    pltpu.sync_copy(att_ref, att_vm)
    pltpu.sync_copy(zbuf, acc_sh.at[pl.ds(row0, ROWS_PER_TILE)])
    plsc.subcore_barrier()
    edges_per_tile = E_PAD // (_NSC * _NTILES)
    n_chunks = edges_per_tile // K_CHUNK

    def chunk(k, _):
        e0 = (cid * (E_PAD // _NSC) + sid * edges_per_tile + k * K_CHUNK)
        _edge_chunk(0, e0, src_ref, dst_ref, xl2_ref, xr2_ref,
                    src_v, dst_v, src_v, dst_v, xl_s, xr_s, contrib,
                    att_vm, acc_sh, sem1, sem2)
        return 0

    lax.fori_loop(0, n_chunks, chunk, 0)
    plsc.subcore_barrier()
    pltpu.sync_copy(acc_sh.at[pl.ds(row0, ROWS_PER_TILE)],
                    out_ref.at[cid].at[pl.ds(row0, ROWS_PER_TILE)])


_l2_edges = functools.partial(
    pl.kernel,
    out_type=jax.ShapeDtypeStruct((_NSC, N_PAD, ACC_W), jnp.float32),
    mesh=_MESH,
    compiler_params=_SC_PARAMS,
    scratch_types=[
        pltpu.VMEM((K_CHUNK,), jnp.int32),
        pltpu.VMEM((K_CHUNK,), jnp.int32),
        pltpu.VMEM((K_CHUNK, C), jnp.float32),
        pltpu.VMEM((K_CHUNK, C), jnp.float32),
        pltpu.VMEM((K_CHUNK, ACC_W), jnp.float32),
        pltpu.VMEM((ROWS_PER_TILE, ACC_W), jnp.float32),
        pltpu.VMEM((C, 16), jnp.float32),
        pltpu.VMEM_SHARED((N_PAD, ACC_W), jnp.float32),
        pltpu.SemaphoreType.DMA,
        pltpu.SemaphoreType.DMA,
    ],
)(_l2_body)


def kernel(x, edge_index, Wl1, Wr1, att1, b1, Wl2, Wr2, att2, b2):
    ei = edge_index.astype(jnp.int32)
    loop_idx = jnp.arange(N_NODES, dtype=jnp.int32)
    pad_e = E_PAD - E_TOT
    src = jnp.concatenate(
        [ei[0], loop_idx, jnp.zeros((pad_e,), jnp.int32)])
    dst = jnp.concatenate(
        [ei[1], loop_idx, jnp.full((pad_e,), N_NODES, jnp.int32)])

    x_pad = jnp.pad(x, ((0, N_PAD - N_NODES), (0, 0)))
    wl3 = Wl1.reshape(H1, C, F_IN)
    wr3 = Wr1.reshape(H1, C, F_IN)

    xlT, xrT = _proj1(x_pad, wl3, wr3)
    att1_bc = jnp.tile(att1.reshape(D1, 1), (1, 16))
    att2_bc = jnp.tile(att2.reshape(C, 1), (1, 16))
    acc1 = _l1_edges(xlT.reshape(H1 * N_PAD, C),
                     xrT.reshape(H1 * N_PAD, C),
                     att1_bc, src, dst)
    xl2, xr2 = _mid(acc1, b1.reshape(1, D1), Wl2, Wr2)
    acc2 = _l2_edges(xl2, xr2, att2_bc, src, dst)
    return _final(acc2, b2.reshape(1, C))


# speedup vs baseline: 10.2375x; 1.1321x over previous
import functools

import jax
import jax.numpy as jnp
from jax import lax
from jax.experimental import pallas as pl
from jax.experimental.pallas import tpu as pltpu
from jax.experimental.pallas import tpu_sc as plsc

N_NODES = 10000
N_PAD = 10112
F_IN = 128
H1, C = 8, 32
D1 = H1 * C
E_RAW = 320000
E_TOT = E_RAW + N_NODES
K_CHUNK = 512
E_PAD = 344064
ACC_W = 48
ROWS_PER_TILE = N_PAD // 16

_NSC = 2
_NTILES = 16

_Z16 = None


_R1 = 2528
_R3 = 2000


def _proj1_body(x_ref, wl_ref, wr_ref, xl_ref, xr_ref):
    xb = x_ref[...]
    dn = (((1,), (1,)), ((), ()))
    xl_ref[0] = lax.dot_general(xb, wl_ref[0], dn,
                                preferred_element_type=jnp.float32)
    xr_ref[0] = lax.dot_general(xb, wr_ref[0], dn,
                                preferred_element_type=jnp.float32)


def _proj1(x_pad, wl3, wr3):
    grid = (H1, N_PAD // _R1)
    return pl.pallas_call(
        _proj1_body,
        grid=grid,
        in_specs=[
            pl.BlockSpec((_R1, F_IN), lambda h, r: (r, 0)),
            pl.BlockSpec((1, C, F_IN), lambda h, r: (h, 0, 0)),
            pl.BlockSpec((1, C, F_IN), lambda h, r: (h, 0, 0)),
        ],
        out_specs=[
            pl.BlockSpec((1, _R1, C), lambda h, r: (h, r, 0)),
            pl.BlockSpec((1, _R1, C), lambda h, r: (h, r, 0)),
        ],
        out_shape=[
            jax.ShapeDtypeStruct((H1, N_PAD, C), jnp.float32),
            jax.ShapeDtypeStruct((H1, N_PAD, C), jnp.float32),
        ],
    )(x_pad, wl3, wr3)


def _mid_body(acc_ref, b1_ref, wl2_ref, wr2_ref, xl2_ref, xr2_ref):
    parts = []
    for h in range(H1):
        num = acc_ref[h, :, 0:C]
        den = acc_ref[h, :, C:C + 1] + 1e-16
        parts.append(num / den)
    h1 = jnp.concatenate(parts, axis=1) + b1_ref[...]
    h1 = jax.nn.gelu(h1)
    dn = (((1,), (1,)), ((), ()))
    xl2_ref[...] = lax.dot_general(h1, wl2_ref[...], dn,
                                   preferred_element_type=jnp.float32)
    xr2_ref[...] = lax.dot_general(h1, wr2_ref[...], dn,
                                   preferred_element_type=jnp.float32)


def _mid(acc1, b1_2d, wl2, wr2):
    grid = (N_PAD // _R1,)
    return pl.pallas_call(
        _mid_body,
        grid=grid,
        in_specs=[
            pl.BlockSpec((H1, _R1, ACC_W), lambda r: (0, r, 0)),
            pl.BlockSpec((1, D1), lambda r: (0, 0)),
            pl.BlockSpec((C, D1), lambda r: (0, 0)),
            pl.BlockSpec((C, D1), lambda r: (0, 0)),
        ],
        out_specs=[
            pl.BlockSpec((_R1, C), lambda r: (r, 0)),
            pl.BlockSpec((_R1, C), lambda r: (r, 0)),
        ],
        out_shape=[
            jax.ShapeDtypeStruct((N_PAD, C), jnp.float32),
            jax.ShapeDtypeStruct((N_PAD, C), jnp.float32),
        ],
    )(acc1, b1_2d, wl2, wr2)


def _final_body(acc_ref, b2_ref, out_ref):
    a = acc_ref[0] + acc_ref[1]
    num = a[:, 0:C]
    den = a[:, C:C + 1] + 1e-16
    out_ref[...] = num / den + b2_ref[...]


def _final(acc2, b2_2d):
    grid = (N_NODES // _R3,)
    return pl.pallas_call(
        _final_body,
        grid=grid,
        in_specs=[
            pl.BlockSpec((_NSC, _R3, ACC_W), lambda r: (0, r, 0)),
            pl.BlockSpec((1, C), lambda r: (0, 0)),
        ],
        out_specs=pl.BlockSpec((_R3, C), lambda r: (r, 0)),
        out_shape=jax.ShapeDtypeStruct((N_NODES, C), jnp.float32),
    )(acc2, b2_2d)


_MESH = plsc.VectorSubcoreMesh(core_axis_name="c", subcore_axis_name="s")
_SC_PARAMS = pltpu.CompilerParams(use_tc_tiling_on_sc=False,
                                  needs_layout_passes=False)


def _zero_scratch(zbuf, contrib):
    z16 = jnp.zeros((16,), jnp.float32)

    def zb(i, _):
        for j in range(ACC_W // 16):
            zbuf[i, pl.ds(j * 16, 16)] = z16
        return 0

    lax.fori_loop(0, ROWS_PER_TILE, zb, 0)

    def zc(i, _):
        contrib[i, pl.ds(C, 16)] = z16
        return 0

    lax.fori_loop(0, K_CHUNK, zc, 0)


def _edge_chunk(h_off, e0, src_ref, dst_ref, xl_flat, xr_flat,
                src_v, dst_v, srch_v, dsth_v, xl_s, xr_s, contrib,
                att_vm, acc_sh, sem1, sem2):
    pltpu.sync_copy(src_ref.at[pl.ds(e0, K_CHUNK)], src_v)
    pltpu.sync_copy(dst_ref.at[pl.ds(e0, K_CHUNK)], dst_v)

    def mkoff(i, _):
        srch_v[pl.ds(i * 16, 16)] = src_v[pl.ds(i * 16, 16)] + h_off
        dsth_v[pl.ds(i * 16, 16)] = dst_v[pl.ds(i * 16, 16)] + h_off
        return 0

    lax.fori_loop(0, K_CHUNK // 16, mkoff, 0)

    c1 = pltpu.async_copy(xl_flat.at[srch_v], xl_s, sem1)
    c2 = pltpu.async_copy(xr_flat.at[dsth_v], xr_s, sem2)
    c1.wait()
    c2.wait()

    iota = lax.iota(jnp.int32, 16)

    def group(g, _):
        rows = g * 16 + iota
        acc = jnp.zeros((16,), jnp.float32)
        a_vals = []
        for cc in range(C):
            colv = jnp.full((16,), cc, jnp.int32)
            a = plsc.load_gather(xl_s, [rows, colv])
            b = plsc.load_gather(xr_s, [rows, colv])
            s = a + b
            lr = jnp.where(s >= 0.0, s, 0.2 * s)
            acc = acc + att_vm[cc] * lr
            a_vals.append(a)
        ev = jnp.exp(jnp.minimum(acc, 45.0))
        plsc.store_scatter(contrib, [rows, jnp.full((16,), C, jnp.int32)], ev)
        for cc in range(C):
            colv = jnp.full((16,), cc, jnp.int32)
            plsc.store_scatter(contrib, [rows, colv], ev * a_vals[cc])
        return 0

    lax.fori_loop(0, K_CHUNK // 16, group, 0)

    pltpu.sync_copy(contrib, acc_sh.at[dst_v], add=True)


def _l1_body(xl_flat, xr_flat, att_ref, src_ref, dst_ref, out_ref,
             src_v, dst_v, srch_v, dsth_v, xl_s, xr_s, contrib, zbuf,
             att_vm, acc_sh, sem1, sem2):
    cid = lax.axis_index("c")
    sid = lax.axis_index("s")
    _zero_scratch(zbuf, contrib)
    row0 = sid * ROWS_PER_TILE
    edges_per_tile = E_PAD // _NTILES
    n_chunks = edges_per_tile // K_CHUNK

    def head(hh, _):
        h = cid * (H1 // _NSC) + hh
        pltpu.sync_copy(att_ref.at[pl.ds(h * C, C)], att_vm)
        pltpu.sync_copy(zbuf, acc_sh.at[pl.ds(row0, ROWS_PER_TILE)])
        plsc.subcore_barrier()
        h_off = h * N_PAD

        def chunk(k, _):
            e0 = sid * edges_per_tile + k * K_CHUNK
            _edge_chunk(h_off, e0, src_ref, dst_ref, xl_flat, xr_flat,
                        src_v, dst_v, srch_v, dsth_v, xl_s, xr_s, contrib,
                        att_vm, acc_sh, sem1, sem2)
            return 0

        lax.fori_loop(0, n_chunks, chunk, 0)
        plsc.subcore_barrier()
        pltpu.sync_copy(acc_sh.at[pl.ds(row0, ROWS_PER_TILE)],
                        out_ref.at[h].at[pl.ds(row0, ROWS_PER_TILE)])
        return 0

    lax.fori_loop(0, H1 // _NSC, head, 0)


_l1_edges = functools.partial(
    pl.kernel,
    out_type=jax.ShapeDtypeStruct((H1, N_PAD, ACC_W), jnp.float32),
    mesh=_MESH,
    compiler_params=_SC_PARAMS,
    scratch_types=[
        pltpu.VMEM((K_CHUNK,), jnp.int32),
        pltpu.VMEM((K_CHUNK,), jnp.int32),
        pltpu.VMEM((K_CHUNK,), jnp.int32),
        pltpu.VMEM((K_CHUNK,), jnp.int32),
        pltpu.VMEM((K_CHUNK, C), jnp.float32),
        pltpu.VMEM((K_CHUNK, C), jnp.float32),
        pltpu.VMEM((K_CHUNK, ACC_W), jnp.float32),
        pltpu.VMEM((ROWS_PER_TILE, ACC_W), jnp.float32),
        pltpu.VMEM((C, 16), jnp.float32),
        pltpu.VMEM_SHARED((N_PAD, ACC_W), jnp.float32),
        pltpu.SemaphoreType.DMA,
        pltpu.SemaphoreType.DMA,
    ],
)(_l1_body)


def _l2_body(xl2_ref, xr2_ref, att_ref, src_ref, dst_ref, out_ref,
             src_v, dst_v, xl_s, xr_s, contrib, zbuf,
             att_vm, acc_sh, sem1, sem2):
    cid = lax.axis_index("c")
    sid = lax.axis_index("s")
    _zero_scratch(zbuf, contrib)
    row0 = sid * ROWS_PER_TILE
    pltpu.sync_copy(att_ref, att_vm)
    pltpu.sync_copy(zbuf, acc_sh.at[pl.ds(row0, ROWS_PER_TILE)])
    plsc.subcore_barrier()
    edges_per_tile = E_PAD // (_NSC * _NTILES)
    n_chunks = edges_per_tile // K_CHUNK

    def chunk(k, _):
        e0 = (cid * (E_PAD // _NSC) + sid * edges_per_tile + k * K_CHUNK)
        _edge_chunk(0, e0, src_ref, dst_ref, xl2_ref, xr2_ref,
                    src_v, dst_v, src_v, dst_v, xl_s, xr_s, contrib,
                    att_vm, acc_sh, sem1, sem2)
        return 0

    lax.fori_loop(0, n_chunks, chunk, 0)
    plsc.subcore_barrier()
    pltpu.sync_copy(acc_sh.at[pl.ds(row0, ROWS_PER_TILE)],
                    out_ref.at[cid].at[pl.ds(row0, ROWS_PER_TILE)])


_l2_edges = functools.partial(
    pl.kernel,
    out_type=jax.ShapeDtypeStruct((_NSC, N_PAD, ACC_W), jnp.float32),
    mesh=_MESH,
    compiler_params=_SC_PARAMS,
    scratch_types=[
        pltpu.VMEM((K_CHUNK,), jnp.int32),
        pltpu.VMEM((K_CHUNK,), jnp.int32),
        pltpu.VMEM((K_CHUNK, C), jnp.float32),
        pltpu.VMEM((K_CHUNK, C), jnp.float32),
        pltpu.VMEM((K_CHUNK, ACC_W), jnp.float32),
        pltpu.VMEM((ROWS_PER_TILE, ACC_W), jnp.float32),
        pltpu.VMEM((C, 16), jnp.float32),
        pltpu.VMEM_SHARED((N_PAD, ACC_W), jnp.float32),
        pltpu.SemaphoreType.DMA,
        pltpu.SemaphoreType.DMA,
    ],
)(_l2_body)


def kernel(x, edge_index, Wl1, Wr1, att1, b1, Wl2, Wr2, att2, b2):
    ei = edge_index.astype(jnp.int32)
    loop_idx = jnp.arange(N_NODES, dtype=jnp.int32)
    pad_e = E_PAD - E_TOT
    src = jnp.concatenate(
        [ei[0], loop_idx, jnp.zeros((pad_e,), jnp.int32)])
    dst = jnp.concatenate(
        [ei[1], loop_idx, jnp.full((pad_e,), N_NODES, jnp.int32)])

    x_pad = jnp.pad(x, ((0, N_PAD - N_NODES), (0, 0)))
    wl3 = Wl1.reshape(H1, C, F_IN)
    wr3 = Wr1.reshape(H1, C, F_IN)

    xlT, xrT = _proj1(x_pad, wl3, wr3)
    att1_bc = jnp.tile(att1.reshape(D1, 1), (1, 16))
    att2_bc = jnp.tile(att2.reshape(C, 1), (1, 16))
    acc1 = _l1_edges(xlT.reshape(H1 * N_PAD, C),
                     xrT.reshape(H1 * N_PAD, C),
                     att1_bc, src, dst)
    xl2, xr2 = _mid(acc1, b1.reshape(1, D1), Wl2, Wr2)
    acc2 = _l2_edges(xl2, xr2, att2_bc, src, dst)
    return _final(acc2, b2.reshape(1, C))


# speedup vs baseline: 13.5385x; 1.3224x over previous
import functools

import jax
import jax.numpy as jnp
from jax import lax
from jax.experimental import pallas as pl
from jax.experimental.pallas import tpu as pltpu
from jax.experimental.pallas import tpu_sc as plsc

N_NODES = 10000
N_PAD = 10112
F_IN = 128
H1, C = 8, 32
D1 = H1 * C
E_RAW = 320000
E_TOT = E_RAW + N_NODES
K_CHUNK = 384
E_PAD1 = 344064
E_PAD2 = 344064
ACC_W = 48
ROWS_PER_TILE = N_PAD // 16
ZR = 79

_NSC = 2
_NTILES = 16


_R1 = 2528
_R3 = 2000


def _proj1_body(x_ref, wl_ref, wr_ref, xl_ref, xr_ref):
    xb = x_ref[...]
    dn = (((1,), (1,)), ((), ()))
    xl_ref[0] = lax.dot_general(xb, wl_ref[0], dn,
                                preferred_element_type=jnp.float32)
    xr_ref[0] = lax.dot_general(xb, wr_ref[0], dn,
                                preferred_element_type=jnp.float32)


def _proj1(x_pad, wl3, wr3):
    grid = (H1, N_PAD // _R1)
    return pl.pallas_call(
        _proj1_body,
        grid=grid,
        in_specs=[
            pl.BlockSpec((_R1, F_IN), lambda h, r: (r, 0)),
            pl.BlockSpec((1, C, F_IN), lambda h, r: (h, 0, 0)),
            pl.BlockSpec((1, C, F_IN), lambda h, r: (h, 0, 0)),
        ],
        out_specs=[
            pl.BlockSpec((1, _R1, C), lambda h, r: (h, r, 0)),
            pl.BlockSpec((1, _R1, C), lambda h, r: (h, r, 0)),
        ],
        out_shape=[
            jax.ShapeDtypeStruct((H1, N_PAD, C), jnp.float32),
            jax.ShapeDtypeStruct((H1, N_PAD, C), jnp.float32),
        ],
    )(x_pad, wl3, wr3)


def _mid_body(acc_ref, b1_ref, wl2_ref, wr2_ref, xl2_ref, xr2_ref):
    parts = []
    for h in range(H1):
        num = acc_ref[h, :, 0:C]
        den = acc_ref[h, :, C:C + 1] + 1e-16
        parts.append(num / den)
    h1 = jnp.concatenate(parts, axis=1) + b1_ref[...]
    h1 = jax.nn.gelu(h1)
    dn = (((1,), (1,)), ((), ()))
    xl2_ref[...] = lax.dot_general(h1, wl2_ref[...], dn,
                                   preferred_element_type=jnp.float32)
    xr2_ref[...] = lax.dot_general(h1, wr2_ref[...], dn,
                                   preferred_element_type=jnp.float32)


def _mid(acc1, b1_2d, wl2, wr2):
    grid = (N_PAD // _R1,)
    return pl.pallas_call(
        _mid_body,
        grid=grid,
        in_specs=[
            pl.BlockSpec((H1, _R1, ACC_W), lambda r: (0, r, 0)),
            pl.BlockSpec((1, D1), lambda r: (0, 0)),
            pl.BlockSpec((C, D1), lambda r: (0, 0)),
            pl.BlockSpec((C, D1), lambda r: (0, 0)),
        ],
        out_specs=[
            pl.BlockSpec((_R1, C), lambda r: (r, 0)),
            pl.BlockSpec((_R1, C), lambda r: (r, 0)),
        ],
        out_shape=[
            jax.ShapeDtypeStruct((N_PAD, C), jnp.float32),
            jax.ShapeDtypeStruct((N_PAD, C), jnp.float32),
        ],
    )(acc1, b1_2d, wl2, wr2)


def _final_body(acc_ref, b2_ref, out_ref):
    a = acc_ref[0] + acc_ref[1]
    num = a[:, 0:C]
    den = a[:, C:C + 1] + 1e-16
    out_ref[...] = num / den + b2_ref[...]


def _final(acc2, b2_2d):
    grid = (N_NODES // _R3,)
    return pl.pallas_call(
        _final_body,
        grid=grid,
        in_specs=[
            pl.BlockSpec((_NSC, _R3, ACC_W), lambda r: (0, r, 0)),
            pl.BlockSpec((1, C), lambda r: (0, 0)),
        ],
        out_specs=pl.BlockSpec((_R3, C), lambda r: (r, 0)),
        out_shape=jax.ShapeDtypeStruct((N_NODES, C), jnp.float32),
    )(acc2, b2_2d)


_MESH = plsc.VectorSubcoreMesh(core_axis_name="c", subcore_axis_name="s")
_SC_PARAMS = pltpu.CompilerParams(use_tc_tiling_on_sc=False,
                                  needs_layout_passes=False)


def _zero_scratch(zbuf):
    z16 = jnp.zeros((16,), jnp.float32)

    def zb(i, _):
        for j in range(ACC_W // 16):
            zbuf[i, pl.ds(j * 16, 16)] = z16
        return 0

    lax.fori_loop(0, ZR, zb, 0)


class _Bufs:
    def __init__(self, src_v, dst_v, srch_v, dsth_v, xl_s, xr_s, contrib,
                 dstS, zbuf, att_vm, acc_sh, gsl, gsr, ssem):
        self.src_v, self.dst_v = src_v, dst_v
        self.srch_v, self.dsth_v = srch_v, dsth_v
        self.xl_s, self.xr_s = xl_s, xr_s
        self.contrib, self.dstS = contrib, dstS
        self.zbuf, self.att_vm, self.acc_sh = zbuf, att_vm, acc_sh
        self.gsl, self.gsr, self.ssem = gsl, gsr, ssem


def _start_chunk(B, b, e0, h_off, src_ref, dst_ref, xl_flat, xr_flat):
    pltpu.sync_copy(src_ref.at[pl.ds(e0, K_CHUNK)], B.src_v.at[b])
    pltpu.sync_copy(dst_ref.at[pl.ds(e0, K_CHUNK)], B.dst_v.at[b])

    def mkoff(i, _):
        B.srch_v[b, pl.ds(i * 16, 16)] = B.src_v[b, pl.ds(i * 16, 16)] + h_off
        B.dsth_v[b, pl.ds(i * 16, 16)] = B.dst_v[b, pl.ds(i * 16, 16)] + h_off
        return 0

    lax.fori_loop(0, K_CHUNK // 16, mkoff, 0)
    pltpu.async_copy(xl_flat.at[B.srch_v.at[b]], B.xl_s.at[b], B.gsl.at[b])
    pltpu.async_copy(xr_flat.at[B.dsth_v.at[b]], B.xr_s.at[b], B.gsr.at[b])


def _wait_gathers(B, b, xl_flat, xr_flat):
    pltpu.make_async_copy(xl_flat.at[B.srch_v.at[b]], B.xl_s.at[b],
                          B.gsl.at[b]).wait()
    pltpu.make_async_copy(xr_flat.at[B.dsth_v.at[b]], B.xr_s.at[b],
                          B.gsr.at[b]).wait()


def _wait_scatter(B, b):
    pltpu.make_async_copy(B.contrib.at[b], B.acc_sh.at[B.dstS.at[b]],
                          B.ssem.at[b]).wait()


def _compute_chunk(B, b):
    def cpd(i, _):
        B.dstS[b, pl.ds(i * 16, 16)] = B.dst_v[b, pl.ds(i * 16, 16)]
        return 0

    lax.fori_loop(0, K_CHUNK // 16, cpd, 0)
    iota = lax.iota(jnp.int32, 16)
    xl_b = B.xl_s.at[b]
    xr_b = B.xr_s.at[b]
    ct_b = B.contrib.at[b]

    def group(g, _):
        rows = g * 16 + iota
        acc = jnp.zeros((16,), jnp.float32)
        a_vals = []
        for cc in range(C):
            colv = jnp.full((16,), cc, jnp.int32)
            a = plsc.load_gather(xl_b, [rows, colv])
            bb = plsc.load_gather(xr_b, [rows, colv])
            s = a + bb
            lr = jnp.where(s >= 0.0, s, 0.2 * s)
            acc = acc + B.att_vm[cc] * lr
            a_vals.append(a)
        ev = jnp.exp(jnp.minimum(acc, 45.0))
        plsc.store_scatter(ct_b, [rows, jnp.full((16,), C, jnp.int32)], ev)
        for cc in range(C):
            colv = jnp.full((16,), cc, jnp.int32)
            plsc.store_scatter(ct_b, [rows, colv], ev * a_vals[cc])
        return 0

    lax.fori_loop(0, K_CHUNK // 16, group, 0)
    pltpu.async_copy(B.contrib.at[b], B.acc_sh.at[B.dstS.at[b]],
                     B.ssem.at[b], add=True)


def _edge_pipeline(B, e_base, n_chunks, h_off, src_ref, dst_ref,
                   xl_flat, xr_flat):
    def start(k, b):
        _start_chunk(B, b, e_base + k * K_CHUNK, h_off,
                     src_ref, dst_ref, xl_flat, xr_flat)

    start(0, 0)
    for k in (0, 1):
        b = k % 2
        _wait_gathers(B, b, xl_flat, xr_flat)
        start(k + 1, 1 - b)
        _compute_chunk(B, b)

    def pair(p, _):
        for b in (0, 1):
            k = 2 * p + b
            _wait_gathers(B, b, xl_flat, xr_flat)

            @pl.when(k + 1 < n_chunks)
            def _():
                start(k + 1, 1 - b)

            _wait_scatter(B, b)
            _compute_chunk(B, b)
        return 0

    lax.fori_loop(1, n_chunks // 2, pair, 0)
    _wait_scatter(B, 0)
    _wait_scatter(B, 1)


def _l1_body(xl_flat, xr_flat, att_ref, src_ref, dst_ref, out_ref, *scr):
    B = _Bufs(*scr)
    cid = lax.axis_index("c")
    sid = lax.axis_index("s")
    _zero_scratch(B.zbuf)
    row0 = sid * ROWS_PER_TILE
    edges_per_tile = E_PAD1 // _NTILES
    n_chunks = edges_per_tile // K_CHUNK

    def head(hh, _):
        h = cid * (H1 // _NSC) + hh
        pltpu.sync_copy(att_ref.at[pl.ds(h * C, C)], B.att_vm)
        for j in range(8):
            pltpu.sync_copy(B.zbuf,
                            B.acc_sh.at[pl.ds(row0 + j * ZR, ZR)])
        plsc.subcore_barrier()
        _edge_pipeline(B, sid * edges_per_tile, n_chunks, h * N_PAD,
                       src_ref, dst_ref, xl_flat, xr_flat)
        plsc.subcore_barrier()
        pltpu.sync_copy(B.acc_sh.at[pl.ds(row0, ROWS_PER_TILE)],
                        out_ref.at[h].at[pl.ds(row0, ROWS_PER_TILE)])
        return 0

    lax.fori_loop(0, H1 // _NSC, head, 0)


def _l2_body(xl2_ref, xr2_ref, att_ref, src_ref, dst_ref, out_ref, *scr):
    B = _Bufs(*scr)
    cid = lax.axis_index("c")
    sid = lax.axis_index("s")
    _zero_scratch(B.zbuf)
    row0 = sid * ROWS_PER_TILE
    pltpu.sync_copy(att_ref, B.att_vm)
    for j in range(8):
        pltpu.sync_copy(B.zbuf, B.acc_sh.at[pl.ds(row0 + j * ZR, ZR)])
    plsc.subcore_barrier()
    edges_per_tile = E_PAD2 // (_NSC * _NTILES)
    n_chunks = edges_per_tile // K_CHUNK
    e_base = cid * (E_PAD2 // _NSC) + sid * edges_per_tile
    _edge_pipeline(B, e_base, n_chunks, 0,
                   src_ref, dst_ref, xl2_ref, xr2_ref)
    plsc.subcore_barrier()
    pltpu.sync_copy(B.acc_sh.at[pl.ds(row0, ROWS_PER_TILE)],
                    out_ref.at[cid].at[pl.ds(row0, ROWS_PER_TILE)])


def _sc_scratch(att_rows):
    return [
        pltpu.VMEM((2, K_CHUNK), jnp.int32),
        pltpu.VMEM((2, K_CHUNK), jnp.int32),
        pltpu.VMEM((2, K_CHUNK), jnp.int32),
        pltpu.VMEM((2, K_CHUNK), jnp.int32),
        pltpu.VMEM((2, K_CHUNK, C), jnp.float32),
        pltpu.VMEM((2, K_CHUNK, C), jnp.float32),
        pltpu.VMEM((2, K_CHUNK, ACC_W), jnp.float32),
        pltpu.VMEM((2, K_CHUNK), jnp.int32),
        pltpu.VMEM((ZR, ACC_W), jnp.float32),
        pltpu.VMEM((att_rows, 16), jnp.float32),
        pltpu.VMEM_SHARED((N_PAD, ACC_W), jnp.float32),
        pltpu.SemaphoreType.DMA((2,)),
        pltpu.SemaphoreType.DMA((2,)),
        pltpu.SemaphoreType.DMA((2,)),
    ]


_l1_edges = functools.partial(
    pl.kernel,
    out_type=jax.ShapeDtypeStruct((H1, N_PAD, ACC_W), jnp.float32),
    mesh=_MESH,
    compiler_params=_SC_PARAMS,
    scratch_types=_sc_scratch(C),
)(_l1_body)


_l2_edges = functools.partial(
    pl.kernel,
    out_type=jax.ShapeDtypeStruct((_NSC, N_PAD, ACC_W), jnp.float32),
    mesh=_MESH,
    compiler_params=_SC_PARAMS,
    scratch_types=_sc_scratch(C),
)(_l2_body)


def kernel(x, edge_index, Wl1, Wr1, att1, b1, Wl2, Wr2, att2, b2):
    ei = edge_index.astype(jnp.int32)
    loop_idx = jnp.arange(N_NODES, dtype=jnp.int32)
    pad_e = E_PAD2 - E_TOT
    src = jnp.concatenate(
        [ei[0], loop_idx, jnp.zeros((pad_e,), jnp.int32)])
    dst = jnp.concatenate(
        [ei[1], loop_idx, jnp.full((pad_e,), N_NODES, jnp.int32)])

    x_pad = jnp.pad(x, ((0, N_PAD - N_NODES), (0, 0)))
    wl3 = Wl1.reshape(H1, C, F_IN)
    wr3 = Wr1.reshape(H1, C, F_IN)

    xlT, xrT = _proj1(x_pad, wl3, wr3)
    att1_bc = jnp.tile(att1.reshape(D1, 1), (1, 16))
    att2_bc = jnp.tile(att2.reshape(C, 1), (1, 16))
    acc1 = _l1_edges(xlT.reshape(H1 * N_PAD, C),
                     xrT.reshape(H1 * N_PAD, C),
                     att1_bc, src, dst)
    xl2, xr2 = _mid(acc1, b1.reshape(1, D1), Wl2, Wr2)
    acc2 = _l2_edges(xl2, xr2, att2_bc, src, dst)
    return _final(acc2, b2.reshape(1, C))


# speedup vs baseline: 13.6156x; 1.0057x over previous
import functools

import jax
import jax.numpy as jnp
from jax import lax
from jax.experimental import pallas as pl
from jax.experimental.pallas import tpu as pltpu
from jax.experimental.pallas import tpu_sc as plsc

N_NODES = 10000
N_PAD = 10112
F_IN = 128
H1, C = 8, 32
D1 = H1 * C
E_RAW = 320000
E_TOT = E_RAW + N_NODES
K_CHUNK = 384
E_PAD1 = 344064
E_PAD2 = 344064
ACC_W = 36
ROWS_PER_TILE = N_PAD // 16
ZR = 79

_NSC = 2
_NTILES = 16


_R1 = 2528
_R3 = 2000


def _proj1_body(x_ref, wl_ref, wr_ref, xl_ref, xr_ref):
    xb = x_ref[...]
    dn = (((1,), (1,)), ((), ()))
    xl_ref[0] = lax.dot_general(xb, wl_ref[0], dn,
                                preferred_element_type=jnp.float32)
    xr_ref[0] = lax.dot_general(xb, wr_ref[0], dn,
                                preferred_element_type=jnp.float32)


def _proj1(x_pad, wl3, wr3):
    grid = (H1, N_PAD // _R1)
    return pl.pallas_call(
        _proj1_body,
        grid=grid,
        in_specs=[
            pl.BlockSpec((_R1, F_IN), lambda h, r: (r, 0)),
            pl.BlockSpec((1, C, F_IN), lambda h, r: (h, 0, 0)),
            pl.BlockSpec((1, C, F_IN), lambda h, r: (h, 0, 0)),
        ],
        out_specs=[
            pl.BlockSpec((1, _R1, C), lambda h, r: (h, r, 0)),
            pl.BlockSpec((1, _R1, C), lambda h, r: (h, r, 0)),
        ],
        out_shape=[
            jax.ShapeDtypeStruct((H1, N_PAD, C), jnp.float32),
            jax.ShapeDtypeStruct((H1, N_PAD, C), jnp.float32),
        ],
    )(x_pad, wl3, wr3)


def _mid_body(acc_ref, b1_ref, wl2_ref, wr2_ref, xl2_ref, xr2_ref):
    parts = []
    for h in range(H1):
        num = acc_ref[h, :, 0:C]
        den = acc_ref[h, :, C:C + 1] + 1e-16
        parts.append(num / den)
    h1 = jnp.concatenate(parts, axis=1) + b1_ref[...]
    h1 = jax.nn.gelu(h1)
    dn = (((1,), (1,)), ((), ()))
    xl2_ref[...] = lax.dot_general(h1, wl2_ref[...], dn,
                                   preferred_element_type=jnp.float32)
    xr2_ref[...] = lax.dot_general(h1, wr2_ref[...], dn,
                                   preferred_element_type=jnp.float32)


def _mid(acc1, b1_2d, wl2, wr2):
    grid = (N_PAD // _R1,)
    return pl.pallas_call(
        _mid_body,
        grid=grid,
        in_specs=[
            pl.BlockSpec((H1, _R1, ACC_W), lambda r: (0, r, 0)),
            pl.BlockSpec((1, D1), lambda r: (0, 0)),
            pl.BlockSpec((C, D1), lambda r: (0, 0)),
            pl.BlockSpec((C, D1), lambda r: (0, 0)),
        ],
        out_specs=[
            pl.BlockSpec((_R1, C), lambda r: (r, 0)),
            pl.BlockSpec((_R1, C), lambda r: (r, 0)),
        ],
        out_shape=[
            jax.ShapeDtypeStruct((N_PAD, C), jnp.float32),
            jax.ShapeDtypeStruct((N_PAD, C), jnp.float32),
        ],
    )(acc1, b1_2d, wl2, wr2)


def _final_body(acc_ref, b2_ref, out_ref):
    a = acc_ref[0] + acc_ref[1]
    num = a[:, 0:C]
    den = a[:, C:C + 1] + 1e-16
    out_ref[...] = num / den + b2_ref[...]


def _final(acc2, b2_2d):
    grid = (N_NODES // _R3,)
    return pl.pallas_call(
        _final_body,
        grid=grid,
        in_specs=[
            pl.BlockSpec((_NSC, _R3, ACC_W), lambda r: (0, r, 0)),
            pl.BlockSpec((1, C), lambda r: (0, 0)),
        ],
        out_specs=pl.BlockSpec((_R3, C), lambda r: (r, 0)),
        out_shape=jax.ShapeDtypeStruct((N_NODES, C), jnp.float32),
    )(acc2, b2_2d)


_MESH = plsc.VectorSubcoreMesh(core_axis_name="c", subcore_axis_name="s")
_SC_PARAMS = pltpu.CompilerParams(use_tc_tiling_on_sc=False,
                                  needs_layout_passes=False)


def _zero_scratch(zbuf, contrib):
    z16 = jnp.zeros((16,), jnp.float32)

    def zb(i, _):
        zbuf[i, pl.ds(0, 16)] = z16
        zbuf[i, pl.ds(16, 16)] = z16
        zbuf[i, pl.ds(ACC_W - 16, 16)] = z16
        return 0

    lax.fori_loop(0, ZR, zb, 0)

    def zc(i, _):
        for b in range(2):
            contrib[b, i, pl.ds(ACC_W - 16, 16)] = z16
        return 0

    lax.fori_loop(0, K_CHUNK, zc, 0)


class _Bufs:
    def __init__(self, src_v, dst_v, srch_v, dsth_v, xl_s, xr_s, contrib,
                 dstS, zbuf, att_vm, acc_sh, gsl, gsr, ssem):
        self.src_v, self.dst_v = src_v, dst_v
        self.srch_v, self.dsth_v = srch_v, dsth_v
        self.xl_s, self.xr_s = xl_s, xr_s
        self.contrib, self.dstS = contrib, dstS
        self.zbuf, self.att_vm, self.acc_sh = zbuf, att_vm, acc_sh
        self.gsl, self.gsr, self.ssem = gsl, gsr, ssem


def _start_chunk(B, b, e0, h_off, src_ref, dst_ref, xl_flat, xr_flat):
    pltpu.sync_copy(src_ref.at[pl.ds(e0, K_CHUNK)], B.src_v.at[b])
    pltpu.sync_copy(dst_ref.at[pl.ds(e0, K_CHUNK)], B.dst_v.at[b])

    def mkoff(i, _):
        B.srch_v[b, pl.ds(i * 16, 16)] = B.src_v[b, pl.ds(i * 16, 16)] + h_off
        B.dsth_v[b, pl.ds(i * 16, 16)] = B.dst_v[b, pl.ds(i * 16, 16)] + h_off
        return 0

    lax.fori_loop(0, K_CHUNK // 16, mkoff, 0)
    pltpu.async_copy(xl_flat.at[B.srch_v.at[b]], B.xl_s.at[b], B.gsl.at[b])
    pltpu.async_copy(xr_flat.at[B.dsth_v.at[b]], B.xr_s.at[b], B.gsr.at[b])


def _wait_gathers(B, b, xl_flat, xr_flat):
    pltpu.make_async_copy(xl_flat.at[B.srch_v.at[b]], B.xl_s.at[b],
                          B.gsl.at[b]).wait()
    pltpu.make_async_copy(xr_flat.at[B.dsth_v.at[b]], B.xr_s.at[b],
                          B.gsr.at[b]).wait()


def _wait_scatter(B, b):
    pltpu.make_async_copy(B.contrib.at[b], B.acc_sh.at[B.dstS.at[b]],
                          B.ssem.at[b]).wait()


def _compute_chunk(B, b):
    def cpd(i, _):
        B.dstS[b, pl.ds(i * 16, 16)] = B.dst_v[b, pl.ds(i * 16, 16)]
        return 0

    lax.fori_loop(0, K_CHUNK // 16, cpd, 0)
    iota = lax.iota(jnp.int32, 16)
    xl_b = B.xl_s.at[b]
    xr_b = B.xr_s.at[b]
    ct_b = B.contrib.at[b]

    def group(g, _):
        rows = g * 16 + iota
        acc = jnp.zeros((16,), jnp.float32)
        a_vals = []
        for cc in range(C):
            colv = jnp.full((16,), cc, jnp.int32)
            a = plsc.load_gather(xl_b, [rows, colv])
            bb = plsc.load_gather(xr_b, [rows, colv])
            s = a + bb
            lr = jnp.where(s >= 0.0, s, 0.2 * s)
            acc = acc + B.att_vm[cc] * lr
            a_vals.append(a)
        ev = jnp.exp(jnp.minimum(acc, 45.0))
        plsc.store_scatter(ct_b, [rows, jnp.full((16,), C, jnp.int32)], ev)
        for cc in range(C):
            colv = jnp.full((16,), cc, jnp.int32)
            plsc.store_scatter(ct_b, [rows, colv], ev * a_vals[cc])
        return 0

    lax.fori_loop(0, K_CHUNK // 16, group, 0)
    pltpu.async_copy(B.contrib.at[b], B.acc_sh.at[B.dstS.at[b]],
                     B.ssem.at[b], add=True)


def _edge_pipeline(B, e_base, n_chunks, h_off, src_ref, dst_ref,
                   xl_flat, xr_flat):
    def start(k, b):
        _start_chunk(B, b, e_base + k * K_CHUNK, h_off,
                     src_ref, dst_ref, xl_flat, xr_flat)

    start(0, 0)
    for k in (0, 1):
        b = k % 2
        _wait_gathers(B, b, xl_flat, xr_flat)
        start(k + 1, 1 - b)
        _compute_chunk(B, b)

    def pair(p, _):
        for b in (0, 1):
            k = 2 * p + b
            _wait_gathers(B, b, xl_flat, xr_flat)

            @pl.when(k + 1 < n_chunks)
            def _():
                start(k + 1, 1 - b)

            _wait_scatter(B, b)
            _compute_chunk(B, b)
        return 0

    lax.fori_loop(1, n_chunks // 2, pair, 0)
    _wait_scatter(B, 0)
    _wait_scatter(B, 1)


def _l1_body(xl_flat, xr_flat, att_ref, src_ref, dst_ref, out_ref, *scr):
    B = _Bufs(*scr)
    cid = lax.axis_index("c")
    sid = lax.axis_index("s")
    _zero_scratch(B.zbuf, B.contrib)
    row0 = sid * ROWS_PER_TILE
    edges_per_tile = E_PAD1 // _NTILES
    n_chunks = edges_per_tile // K_CHUNK

    def head(hh, _):
        h = cid * (H1 // _NSC) + hh
        pltpu.sync_copy(att_ref.at[pl.ds(h * C, C)], B.att_vm)
        for j in range(8):
            pltpu.sync_copy(B.zbuf,
                            B.acc_sh.at[pl.ds(row0 + j * ZR, ZR)])
        plsc.subcore_barrier()
        _edge_pipeline(B, sid * edges_per_tile, n_chunks, h * N_PAD,
                       src_ref, dst_ref, xl_flat, xr_flat)
        plsc.subcore_barrier()
        pltpu.sync_copy(B.acc_sh.at[pl.ds(row0, ROWS_PER_TILE)],
                        out_ref.at[h].at[pl.ds(row0, ROWS_PER_TILE)])
        return 0

    lax.fori_loop(0, H1 // _NSC, head, 0)


def _l2_body(xl2_ref, xr2_ref, att_ref, src_ref, dst_ref, out_ref, *scr):
    B = _Bufs(*scr)
    cid = lax.axis_index("c")
    sid = lax.axis_index("s")
    _zero_scratch(B.zbuf, B.contrib)
    row0 = sid * ROWS_PER_TILE
    pltpu.sync_copy(att_ref, B.att_vm)
    for j in range(8):
        pltpu.sync_copy(B.zbuf, B.acc_sh.at[pl.ds(row0 + j * ZR, ZR)])
    plsc.subcore_barrier()
    edges_per_tile = E_PAD2 // (_NSC * _NTILES)
    n_chunks = edges_per_tile // K_CHUNK
    e_base = cid * (E_PAD2 // _NSC) + sid * edges_per_tile
    _edge_pipeline(B, e_base, n_chunks, 0,
                   src_ref, dst_ref, xl2_ref, xr2_ref)
    plsc.subcore_barrier()
    pltpu.sync_copy(B.acc_sh.at[pl.ds(row0, ROWS_PER_TILE)],
                    out_ref.at[cid].at[pl.ds(row0, ROWS_PER_TILE)])


def _sc_scratch(att_rows):
    return [
        pltpu.VMEM((2, K_CHUNK), jnp.int32),
        pltpu.VMEM((2, K_CHUNK), jnp.int32),
        pltpu.VMEM((2, K_CHUNK), jnp.int32),
        pltpu.VMEM((2, K_CHUNK), jnp.int32),
        pltpu.VMEM((2, K_CHUNK, C), jnp.float32),
        pltpu.VMEM((2, K_CHUNK, C), jnp.float32),
        pltpu.VMEM((2, K_CHUNK, ACC_W), jnp.float32),
        pltpu.VMEM((2, K_CHUNK), jnp.int32),
        pltpu.VMEM((ZR, ACC_W), jnp.float32),
        pltpu.VMEM((att_rows, 16), jnp.float32),
        pltpu.VMEM_SHARED((N_PAD, ACC_W), jnp.float32),
        pltpu.SemaphoreType.DMA((2,)),
        pltpu.SemaphoreType.DMA((2,)),
        pltpu.SemaphoreType.DMA((2,)),
    ]


_l1_edges = functools.partial(
    pl.kernel,
    out_type=jax.ShapeDtypeStruct((H1, N_PAD, ACC_W), jnp.float32),
    mesh=_MESH,
    compiler_params=_SC_PARAMS,
    scratch_types=_sc_scratch(C),
)(_l1_body)


_l2_edges = functools.partial(
    pl.kernel,
    out_type=jax.ShapeDtypeStruct((_NSC, N_PAD, ACC_W), jnp.float32),
    mesh=_MESH,
    compiler_params=_SC_PARAMS,
    scratch_types=_sc_scratch(C),
)(_l2_body)


def kernel(x, edge_index, Wl1, Wr1, att1, b1, Wl2, Wr2, att2, b2):
    ei = edge_index.astype(jnp.int32)
    loop_idx = jnp.arange(N_NODES, dtype=jnp.int32)
    pad_e = E_PAD2 - E_TOT
    src = jnp.concatenate(
        [ei[0], loop_idx, jnp.zeros((pad_e,), jnp.int32)])
    dst = jnp.concatenate(
        [ei[1], loop_idx, jnp.full((pad_e,), N_NODES, jnp.int32)])

    x_pad = jnp.pad(x, ((0, N_PAD - N_NODES), (0, 0)))
    wl3 = Wl1.reshape(H1, C, F_IN)
    wr3 = Wr1.reshape(H1, C, F_IN)

    xlT, xrT = _proj1(x_pad, wl3, wr3)
    att1_bc = jnp.tile(att1.reshape(D1, 1), (1, 16))
    att2_bc = jnp.tile(att2.reshape(C, 1), (1, 16))
    acc1 = _l1_edges(xlT.reshape(H1 * N_PAD, C),
                     xrT.reshape(H1 * N_PAD, C),
                     att1_bc, src, dst)
    xl2, xr2 = _mid(acc1, b1.reshape(1, D1), Wl2, Wr2)
    acc2 = _l2_edges(xl2, xr2, att2_bc, src, dst)
    return _final(acc2, b2.reshape(1, C))
